# baseline (speedup 1.0000x reference)
import jax
import jax.numpy as jnp
from jax.experimental import pallas as pl
from jax.experimental.pallas import tpu as pltpu

_K = 1024
_D = 64
_BETA = 0.25
_BB = 2
_PW = 2 * _D + 3


def _vq_body(x_ref, emb_ref, xq_ref, codes_ref, loss_ref,
             embm2_ref, e2_ref, p_ref):
    b = pl.program_id(0)

    @pl.when(b == 0)
    def _prep():
        emb = emb_ref[...]
        embm2_ref[...] = emb * -2.0
        e2_ref[...] = jnp.sum(emb * emb, axis=1)[:, None]
        p1 = emb.astype(jnp.bfloat16)
        p2 = (emb - p1.astype(jnp.float32)).astype(jnp.bfloat16)
        kcol = jax.lax.broadcasted_iota(jnp.int32, (_K, 1), 0)
        khi = (kcol // 8).astype(jnp.bfloat16)
        klo = (kcol % 8).astype(jnp.bfloat16)
        ones = jnp.ones((_K, 1), jnp.bfloat16)
        p_ref[...] = jnp.concatenate([p1, p2, khi, klo, ones], axis=1)
        loss_ref[0, 0] = 0.0

    x = jnp.concatenate([x_ref[i] for i in range(_BB)], axis=1)
    s2 = jax.lax.dot_general(
        embm2_ref[...], x, (((1,), (0,)), ((), ())),
        preferred_element_type=jnp.float32)
    d = s2 + e2_ref[...]
    m = jnp.min(d, axis=0)
    oh = (d == m[None, :]).astype(jnp.float32)
    g = jax.lax.dot_general(
        p_ref[...], oh, (((0,), (0,)), ((), ())),
        preferred_element_type=jnp.float32)
    hw = x_ref.shape[2]

    def _finish(xq, codes):
        for i in range(_BB):
            xq_ref[i] = xq[:, i * hw:(i + 1) * hw]
            codes_ref[i, 0] = codes[i * hw:(i + 1) * hw]

    loss_ref[0, 0] += jnp.sum(x * x) + jnp.sum(m)

    xq = g[:_D] + g[_D:2 * _D]
    codes = (g[2 * _D] * 8.0 + g[2 * _D + 1]).astype(jnp.int32)
    _finish(xq, codes)

    count = g[2 * _D + 2]
    tie = jnp.max(count) > 1.5

    @pl.when(tie)
    def _slow():
        kio = jax.lax.broadcasted_iota(jnp.int32, d.shape, 0)
        codes_s = jnp.min(jnp.where(d == m[None, :], kio, _K), axis=0)
        oh2 = (kio == codes_s[None, :]).astype(jnp.bfloat16)
        g2 = jax.lax.dot_general(
            p_ref[...], oh2, (((0,), (0,)), ((), ())),
            preferred_element_type=jnp.float32)
        _finish(g2[:_D] + g2[_D:2 * _D], codes_s)


def kernel(x, emb):
    B, D, H, W = x.shape
    HW = H * W
    xr = x.reshape(B, D, HW)
    xq, codes, loss = pl.pallas_call(
        _vq_body,
        grid=(B // _BB,),
        in_specs=[
            pl.BlockSpec((_BB, D, HW), lambda b: (b, 0, 0)),
            pl.BlockSpec((_K, _D), lambda b: (0, 0)),
        ],
        out_specs=[
            pl.BlockSpec((_BB, D, HW), lambda b: (b, 0, 0)),
            pl.BlockSpec((_BB, 1, HW), lambda b: (b, 0, 0)),
            pl.BlockSpec((1, 1), lambda b: (0, 0), memory_space=pltpu.SMEM),
        ],
        out_shape=[
            jax.ShapeDtypeStruct((B, D, HW), jnp.float32),
            jax.ShapeDtypeStruct((B, 1, HW), jnp.int32),
            jax.ShapeDtypeStruct((1, 1), jnp.float32),
        ],
        scratch_shapes=[
            pltpu.VMEM((_K, _D), jnp.float32),
            pltpu.VMEM((_K, 1), jnp.float32),
            pltpu.VMEM((_K, _PW), jnp.bfloat16),
        ],
    )(xr, emb)
    x_q_st = xq.reshape(B, D, H, W)
    vq_loss = loss[0, 0] * ((1.0 + _BETA) / (B * D * H * W))
    codes_map = codes.reshape(B, H, W)
    return (x_q_st, vq_loss, codes_map)


# speedup vs baseline: 1.0326x; 1.0326x over previous
import jax
import jax.numpy as jnp
from jax.experimental import pallas as pl
from jax.experimental.pallas import tpu as pltpu

_K = 1024
_D = 64
_BETA = 0.25
_BB = 4
_PW = 2 * _D + 3


def _vq_body(x_ref, emb_ref, xq_ref, codes_ref, loss_ref,
             embm2_ref, e2_ref, p_ref):
    b = pl.program_id(0)

    @pl.when(b == 0)
    def _prep():
        emb = emb_ref[...]
        embm2_ref[...] = emb * -2.0
        e2_ref[...] = jnp.sum(emb * emb, axis=1)[:, None]
        p1 = emb.astype(jnp.bfloat16)
        p2 = (emb - p1.astype(jnp.float32)).astype(jnp.bfloat16)
        kcol = jax.lax.broadcasted_iota(jnp.int32, (_K, 1), 0)
        khi = (kcol // 8).astype(jnp.bfloat16)
        klo = (kcol % 8).astype(jnp.bfloat16)
        ones = jnp.ones((_K, 1), jnp.bfloat16)
        p_ref[...] = jnp.concatenate([p1, p2, khi, klo, ones], axis=1)
        loss_ref[0, 0] = 0.0

    x = jnp.concatenate([x_ref[i] for i in range(_BB)], axis=1)
    s2 = jax.lax.dot_general(
        embm2_ref[...], x, (((1,), (0,)), ((), ())),
        preferred_element_type=jnp.float32)
    d = s2 + e2_ref[...]
    m = jnp.min(d, axis=0)
    oh = (d == m[None, :]).astype(jnp.float32)
    g = jax.lax.dot_general(
        p_ref[...], oh, (((0,), (0,)), ((), ())),
        preferred_element_type=jnp.float32)
    hw = x_ref.shape[2]

    def _finish(xq, codes):
        for i in range(_BB):
            xq_ref[i] = xq[:, i * hw:(i + 1) * hw]
            codes_ref[i, 0] = codes[i * hw:(i + 1) * hw]

    loss_ref[0, 0] += jnp.sum(x * x) + jnp.sum(m)

    xq = g[:_D] + g[_D:2 * _D]
    codes = (g[2 * _D] * 8.0 + g[2 * _D + 1]).astype(jnp.int32)
    _finish(xq, codes)

    count = g[2 * _D + 2]
    tie = jnp.max(count) > 1.5

    @pl.when(tie)
    def _slow():
        kio = jax.lax.broadcasted_iota(jnp.int32, d.shape, 0)
        codes_s = jnp.min(jnp.where(d == m[None, :], kio, _K), axis=0)
        oh2 = (kio == codes_s[None, :]).astype(jnp.bfloat16)
        g2 = jax.lax.dot_general(
            p_ref[...], oh2, (((0,), (0,)), ((), ())),
            preferred_element_type=jnp.float32)
        _finish(g2[:_D] + g2[_D:2 * _D], codes_s)


def kernel(x, emb):
    B, D, H, W = x.shape
    HW = H * W
    xr = x.reshape(B, D, HW)
    xq, codes, loss = pl.pallas_call(
        _vq_body,
        grid=(B // _BB,),
        in_specs=[
            pl.BlockSpec((_BB, D, HW), lambda b: (b, 0, 0)),
            pl.BlockSpec((_K, _D), lambda b: (0, 0)),
        ],
        out_specs=[
            pl.BlockSpec((_BB, D, HW), lambda b: (b, 0, 0)),
            pl.BlockSpec((_BB, 1, HW), lambda b: (b, 0, 0)),
            pl.BlockSpec((1, 1), lambda b: (0, 0), memory_space=pltpu.SMEM),
        ],
        out_shape=[
            jax.ShapeDtypeStruct((B, D, HW), jnp.float32),
            jax.ShapeDtypeStruct((B, 1, HW), jnp.int32),
            jax.ShapeDtypeStruct((1, 1), jnp.float32),
        ],
        scratch_shapes=[
            pltpu.VMEM((_K, _D), jnp.float32),
            pltpu.VMEM((_K, 1), jnp.float32),
            pltpu.VMEM((_K, _PW), jnp.bfloat16),
        ],
    )(xr, emb)
    x_q_st = xq.reshape(B, D, H, W)
    vq_loss = loss[0, 0] * ((1.0 + _BETA) / (B * D * H * W))
    codes_map = codes.reshape(B, H, W)
    return (x_q_st, vq_loss, codes_map)
